# baseline (speedup 1.0000x reference)
import functools

import jax
import jax.numpy as jnp
from jax import lax
from jax.experimental import pallas as pl
from jax.experimental.pallas import tpu as pltpu
from jax.experimental.pallas import tpu_sc as plsc

N = 10000
E = 320000
H = 256
HALF = 128
L = 4

NC = 2
NS = 16
CH = 80
EPT = E // NS
CPT = 256
PAD_T = CPT * CH - EPT
EPT_H = E // (NC * NS)
UCPT = 128
PAD_U = UCPT * CH - EPT_H

NP = 10240
ROWS_PT = NP // NS
DR = 2 * CH
NDR = ROWS_PT // DR

SEG = 16
NSEG = CPT // SEG
SEG_U = SEG // 2
NSEG_U = UCPT // SEG_U

NCHUNK_D = EPT_H // CH

BN = 1000

_sc_mesh = plsc.VectorSubcoreMesh(core_axis_name="c", subcore_axis_name="s")


def _mm_body(a_ref, w_ref, o_ref):
    o_ref[...] = lax.dot_general(
        a_ref[...], w_ref[...], (((1,), (1,)), ((), ())),
        preferred_element_type=jnp.float32)


def _mm_bias_body(a_ref, w_ref, b_ref, o_ref):
    o_ref[...] = lax.dot_general(
        a_ref[...], w_ref[...], (((1,), (1,)), ((), ())),
        preferred_element_type=jnp.float32) + b_ref[...]


def _matmul(a, w, bias=None):
    n, k = a.shape
    m = w.shape[0]
    in_specs = [pl.BlockSpec((BN, k), lambda i: (i, 0)),
                pl.BlockSpec((m, k), lambda i: (0, 0))]
    args = [a, w]
    body = _mm_body
    if bias is not None:
        in_specs.append(pl.BlockSpec((1, m), lambda i: (0, 0)))
        args.append(bias.reshape(1, m))
        body = _mm_bias_body
    return pl.pallas_call(
        body,
        grid=(n // BN,),
        in_specs=in_specs,
        out_specs=pl.BlockSpec((BN, m), lambda i: (i, 0)),
        out_shape=jax.ShapeDtypeStruct((n, m), jnp.float32),
    )(*args)


def _conv_body(s2_ref, w_ref, b_ref, xa_ref, s_ref):
    agg = (lax.dot_general(s2_ref[0], w_ref[:, :HALF],
                           (((1,), (1,)), ((), ())),
                           preferred_element_type=jnp.float32)
           + lax.dot_general(s2_ref[1], w_ref[:, HALF:],
                             (((1,), (1,)), ((), ())),
                             preferred_element_type=jnp.float32)
           + b_ref[...])
    xa = jnp.maximum(agg, 0.0)
    xa_ref[...] = xa
    s_ref[...] = jnp.sum(xa * xa, axis=1, keepdims=True)


def _conv(S2, w, b):
    return pl.pallas_call(
        _conv_body,
        grid=(N // BN,),
        in_specs=[pl.BlockSpec((2, BN, HALF), lambda i: (0, i, 0)),
                  pl.BlockSpec((H, H), lambda i: (0, 0)),
                  pl.BlockSpec((1, H), lambda i: (0, 0))],
        out_specs=[pl.BlockSpec((BN, H), lambda i: (i, 0)),
                   pl.BlockSpec((BN, 1), lambda i: (i, 0))],
        out_shape=[jax.ShapeDtypeStruct((N, H), jnp.float32),
                   jax.ShapeDtypeStruct((N, 1), jnp.float32)],
    )(S2, w, b.reshape(1, H))


def _new_h(h_ref, xa_ref, xs_ref, t_ref, s_ref, u0_ref, u1_ref,
           d0_ref, d1_ref):
    xa = xa_ref[...]
    dot = (jnp.sum(xa[:, :HALF] * t_ref[0], axis=1)
           + jnp.sum(xa[:, HALF:] * t_ref[1], axis=1))
    u = u0_ref[:, 0] + u1_ref[:, 0]
    deg = d0_ref[:, 0] + d1_ref[:, 0]
    gsum = deg * s_ref[:, 0] + u - 2.0 * dot
    gamma = jnp.tanh(gsum / (deg + 1e-10))[:, None]
    inv = 1.0 / (1.0 + 2.0 * gamma)
    return (h_ref[...] + gamma * (xa + xs_ref[...])) * inv


def _update_body(h_ref, xa_ref, xs_ref, t_ref, s_ref, u0_ref, u1_ref,
                 d0_ref, d1_ref, o_ref):
    o_ref[...] = _new_h(h_ref, xa_ref, xs_ref, t_ref, s_ref, u0_ref, u1_ref,
                        d0_ref, d1_ref)


def _update_final_body(h_ref, xa_ref, xs_ref, t_ref, s_ref, u0_ref, u1_ref,
                       d0_ref, d1_ref, wfc_ref, bfc_ref, o_ref):
    hb = _new_h(h_ref, xa_ref, xs_ref, t_ref, s_ref, u0_ref, u1_ref,
                d0_ref, d1_ref)
    o_ref[...] = lax.dot_general(
        hb, wfc_ref[...], (((1,), (1,)), ((), ())),
        preferred_element_type=jnp.float32) + bfc_ref[...]


def _update(h, x_agg, x_skip, t2, s, u2, deg2, wfc=None, bfc=None):
    col_spec = pl.BlockSpec((BN, 1), lambda i: (i, 0))
    in_specs = [pl.BlockSpec((BN, H), lambda i: (i, 0)),
                pl.BlockSpec((BN, H), lambda i: (i, 0)),
                pl.BlockSpec((BN, H), lambda i: (i, 0)),
                pl.BlockSpec((2, BN, HALF), lambda i: (0, i, 0)),
                col_spec, col_spec, col_spec, col_spec, col_spec]
    args = [h, x_agg, x_skip, t2, s,
            u2[0, :N, None], u2[1, :N, None],
            deg2[0, :N, None], deg2[1, :N, None]]
    if wfc is None:
        body, d_out = _update_body, H
    else:
        body, d_out = _update_final_body, wfc.shape[0]
        in_specs += [pl.BlockSpec((d_out, H), lambda i: (0, 0)),
                     pl.BlockSpec((1, d_out), lambda i: (0, 0))]
        args += [wfc, bfc.reshape(1, d_out)]
    return pl.pallas_call(
        body,
        grid=(N // BN,),
        in_specs=in_specs,
        out_specs=pl.BlockSpec((BN, d_out), lambda i: (i, 0)),
        out_shape=jax.ShapeDtypeStruct((N, d_out), jnp.float32),
    )(*args)


def _zero_acc(z_hbm, dbuf, acc, r0):
    pltpu.sync_copy(z_hbm, dbuf)
    for j in range(NDR):
        pltpu.sync_copy(dbuf, acc.at[pl.ds(r0 + j * DR, DR)])


def _drain_acc(acc, dbuf, out_hbm, c, r0):
    for j in range(NDR):
        pltpu.sync_copy(acc.at[pl.ds(r0 + j * DR, DR)], dbuf)
        pltpu.sync_copy(dbuf, out_hbm.at[c, pl.ds(r0 + j * DR, DR)])


NGRP = SEG // 2


def _edge_pass(src_hbm, ig_hbm, is_hbm, gidx, sidx, bufs, z_hbm, acc,
               semg, semsc, extra=None):
    def start_g(p, i):
        pltpu.async_copy(src_hbm.at[gidx.at[2 * p]],
                         bufs[i].at[pl.ds(0, CH)], semg[i])
        pltpu.async_copy(src_hbm.at[gidx.at[2 * p + 1]],
                         bufs[i].at[pl.ds(CH, CH)], semg[i])

    def wait_g(i):
        pltpu.make_async_copy(src_hbm.at[gidx.at[0]], bufs[i], semg[i]).wait()

    def start_s(p, i):
        pltpu.async_copy(bufs[i].at[pl.ds(0, CH)],
                         acc.at[sidx.at[2 * p]], semsc[i], add=True)
        pltpu.async_copy(bufs[i].at[pl.ds(CH, CH)],
                         acc.at[sidx.at[2 * p + 1]], semsc[i], add=True)

    def wait_s(i):
        pltpu.make_async_copy(z_hbm, bufs[i], semsc[i]).wait()

    @pl.loop(0, NSEG)
    def _(g):
        pltpu.sync_copy(ig_hbm.at[g], gidx)
        pltpu.sync_copy(is_hbm.at[g], sidx)
        start_g(0, 0)

        @pl.loop(0, NGRP, step=2)
        def _(p2):
            for i in range(2):
                other = 1 - i
                wait_g(i)
                start_s(p2 + i, i)
                if extra is not None:
                    extra(g, p2, i)
                if i == 0:
                    @pl.when((g > 0) | (p2 > 0))
                    def _():
                        wait_s(other)
                else:
                    wait_s(other)

                @pl.when(p2 + i + 1 < NGRP)
                def _():
                    start_g(p2 + i + 1, other)

    wait_s(1)


def _sc_agg(m2r, rowg_r, cols_r, zmat):
    @functools.partial(
        pl.kernel,
        out_type=jax.ShapeDtypeStruct((2, NP, HALF), jnp.float32),
        mesh=_sc_mesh,
        scratch_types=[
            pltpu.VMEM((SEG, CH), jnp.int32),
            pltpu.VMEM((SEG, CH), jnp.int32),
            pltpu.VMEM((2 * CH, HALF), jnp.float32),
            pltpu.VMEM((2 * CH, HALF), jnp.float32),
            pltpu.VMEM_SHARED((NP, HALF), jnp.float32),
            pltpu.SemaphoreType.DMA,
            pltpu.SemaphoreType.DMA,
            pltpu.SemaphoreType.DMA,
            pltpu.SemaphoreType.DMA,
        ],
    )
    def k(m_hbm, rg_hbm, cs_hbm, z_hbm, out_hbm,
          gidx, sidx, b0, b1, acc,
          sg0, sg1, ss0, ss1):
        c = lax.axis_index("c")
        s = lax.axis_index("s")
        r0 = s * ROWS_PT
        _zero_acc(z_hbm, b0, acc, r0)
        plsc.subcore_barrier()
        _edge_pass(m_hbm, rg_hbm.at[c, s], cs_hbm.at[s],
                   gidx, sidx, (b0, b1), z_hbm, acc,
                   (sg0, sg1), (ss0, ss1))
        plsc.subcore_barrier()
        _drain_acc(acc, b0, out_hbm, c, r0)

    return k(m2r, rowg_r, cols_r, zmat)


def _sc_gamma(xar, s_vec, colg_r, rows_r, colu_r, rowu_r, zmat, zvec):
    @functools.partial(
        pl.kernel,
        out_type=[jax.ShapeDtypeStruct((2, NP, HALF), jnp.float32),
                  jax.ShapeDtypeStruct((2, NP), jnp.float32)],
        mesh=_sc_mesh,
        scratch_types=[
            pltpu.VMEM((SEG, CH), jnp.int32),
            pltpu.VMEM((SEG, CH), jnp.int32),
            pltpu.VMEM((SEG_U, CH), jnp.int32),
            pltpu.VMEM((SEG_U, CH), jnp.int32),
            pltpu.VMEM((2 * CH, HALF), jnp.float32),
            pltpu.VMEM((2 * CH, HALF), jnp.float32),
            pltpu.VMEM((CH,), jnp.float32),
            pltpu.VMEM((CH,), jnp.float32),
            pltpu.VMEM((ROWS_PT,), jnp.float32),
            pltpu.VMEM_SHARED((NP, HALF), jnp.float32),
            pltpu.VMEM_SHARED((NP,), jnp.float32),
            pltpu.SemaphoreType.DMA,
            pltpu.SemaphoreType.DMA,
            pltpu.SemaphoreType.DMA,
            pltpu.SemaphoreType.DMA,
            pltpu.SemaphoreType.DMA,
            pltpu.SemaphoreType.DMA,
        ],
    )
    def k(xa_hbm, s_hbm, cg_hbm, rs_hbm, cu_hbm, ru_hbm, z_hbm, zv_hbm,
          t_hbm, u_hbm,
          gidx, sidx, ucidx, uridx, b0, b1, sva, svb, ubuf,
          acc, uacc,
          sg0, sg1, ss0, ss1, su0, su1):
        c = lax.axis_index("c")
        s = lax.axis_index("s")
        r0 = s * ROWS_PT
        _zero_acc(z_hbm, b0, acc, r0)
        pltpu.sync_copy(zv_hbm, ubuf)
        pltpu.sync_copy(ubuf, uacc.at[pl.ds(r0, ROWS_PT)])
        plsc.subcore_barrier()

        svals = (sva, svb)
        semu = (su0, su1)

        def u_start(jj, a):
            pltpu.async_copy(s_hbm.at[ucidx.at[jj]], svals[a], semu[a])

        def u_wait(a):
            pltpu.make_async_copy(s_hbm.at[ucidx.at[0]], svals[a],
                                  semu[a]).wait()

        def u_work(g, p2, i):
            j = p2 + i
            a = i

            @pl.when(p2 == 0)
            def _():
                if a == 0:
                    pltpu.sync_copy(cu_hbm.at[c, s, g], ucidx)
                    pltpu.sync_copy(ru_hbm.at[c, s, g], uridx)
                    u_start(0, 0)

            @pl.when(j + 1 < SEG_U)
            def _():
                u_start(j + 1, 1 - a)

            u_wait(a)
            pltpu.sync_copy(svals[a], uacc.at[uridx.at[j]], add=True)

        _edge_pass(xa_hbm, cg_hbm.at[c, s], rs_hbm.at[s],
                   gidx, sidx, (b0, b1), z_hbm, acc,
                   (sg0, sg1), (ss0, ss1),
                   extra=u_work)
        plsc.subcore_barrier()
        _drain_acc(acc, b0, t_hbm, c, r0)
        pltpu.sync_copy(uacc.at[pl.ds(r0, ROWS_PT)], ubuf)
        pltpu.sync_copy(ubuf, u_hbm.at[c, pl.ds(r0, ROWS_PT)])

    return k(xar, s_vec, colg_r, rows_r, colu_r, rowu_r, zmat, zvec)


def _sc_deg(rowh_r, zvec):
    @functools.partial(
        pl.kernel,
        out_type=jax.ShapeDtypeStruct((2, NP), jnp.float32),
        mesh=_sc_mesh,
        scratch_types=[
            pltpu.VMEM((NCHUNK_D, CH), jnp.int32),
            pltpu.VMEM((CH,), jnp.float32),
            pltpu.VMEM((ROWS_PT,), jnp.float32),
            pltpu.VMEM_SHARED((NP,), jnp.float32),
        ],
    )
    def k(rh_hbm, zv_hbm, out_hbm, uridx, ones, zbuf, dacc):
        c = lax.axis_index("c")
        s = lax.axis_index("s")
        r0 = s * ROWS_PT
        for j in range(CH // 16):
            ones[pl.ds(j * 16, 16)] = jnp.ones((16,), jnp.float32)
        pltpu.sync_copy(zv_hbm, zbuf)
        pltpu.sync_copy(zbuf, dacc.at[pl.ds(r0, ROWS_PT)])
        pltpu.sync_copy(rh_hbm.at[c, s], uridx)
        plsc.subcore_barrier()

        @pl.loop(0, NCHUNK_D)
        def _(kk):
            pltpu.sync_copy(ones, dacc.at[uridx.at[kk]], add=True)

        plsc.subcore_barrier()
        pltpu.sync_copy(dacc.at[pl.ds(r0, ROWS_PT)], zbuf)
        pltpu.sync_copy(zbuf, out_hbm.at[c, pl.ds(r0, ROWS_PT)])

    return k(rowh_r, zvec)


def _pad_axis1(a, pad_vals):
    return jnp.concatenate(
        [a, jnp.broadcast_to(pad_vals, a.shape[:-1] + pad_vals.shape[-1:])],
        axis=-1).astype(jnp.int32)


def kernel(x, edge_index, x0, W_in, W_skip, conv_W, conv_b, W_fc, b_fc):
    row = edge_index[0]
    col = edge_index[1]
    two_c = jnp.arange(2, dtype=jnp.int32).reshape(2, 1, 1)

    pad_g = (jnp.arange(PAD_T, dtype=jnp.int32) * 131) % N
    pad_s = N + (jnp.arange(PAD_T, dtype=jnp.int32) % (NP - N))
    pad_gu = pad_g[:PAD_U]
    pad_su = pad_s[:PAD_U]

    row_t = _pad_axis1(row.reshape(NS, EPT), pad_g)
    col_t = _pad_axis1(col.reshape(NS, EPT), pad_g)
    cols_t = _pad_axis1(col.reshape(NS, EPT), pad_s)
    rows_t = _pad_axis1(row.reshape(NS, EPT), pad_s)

    rowg_r = (2 * row_t[None] + two_c).reshape(2, NS, NSEG, SEG, CH)
    colg_r = (2 * col_t[None] + two_c).reshape(2, NS, NSEG, SEG, CH)
    cols_r = cols_t.reshape(NS, NSEG, SEG, CH)
    rows_r = rows_t.reshape(NS, NSEG, SEG, CH)
    colu_r = _pad_axis1(col.reshape(2, NS, EPT_H), pad_gu).reshape(
        2, NS, NSEG_U, SEG_U, CH)
    rowu_r = _pad_axis1(row.reshape(2, NS, EPT_H), pad_su).reshape(
        2, NS, NSEG_U, SEG_U, CH)

    zmat = jnp.zeros((DR, HALF), jnp.float32)
    zvec = jnp.zeros((ROWS_PT,), jnp.float32)

    h = _matmul(x, W_in)
    h0m = _matmul(x0, W_in)
    x_skip = _matmul(h0m, W_skip)
    deg2 = _sc_deg(row.reshape(2, NS, NCHUNK_D, CH), zvec)

    out = None
    for l in range(L):
        S2 = _sc_agg(h.reshape(2 * N, HALF), rowg_r, cols_r, zmat)
        x_agg, s = _conv(S2, conv_W[l], conv_b[l])
        t2, u2 = _sc_gamma(x_agg.reshape(2 * N, HALF), s.reshape(N),
                           colg_r, rows_r, colu_r, rowu_r, zmat, zvec)
        if l < L - 1:
            h = _update(h, x_agg, x_skip, t2, s, u2, deg2)
        else:
            out = _update(h, x_agg, x_skip, t2, s, u2, deg2, W_fc, b_fc)

    return out


# speedup vs baseline: 1.1412x; 1.1412x over previous
import functools

import jax
import jax.numpy as jnp
from jax import lax
from jax.experimental import pallas as pl
from jax.experimental.pallas import tpu as pltpu
from jax.experimental.pallas import tpu_sc as plsc

N = 10000
E = 320000
H = 256
HALF = 128
L = 4

NC = 2
NS = 16
CH = 80
EPT = E // NS
CPT = 256
PAD_T = CPT * CH - EPT
EPT_H = E // (NC * NS)
UCPT = 128
PAD_U = UCPT * CH - EPT_H

NP = 10240
ROWS_PT = NP // NS
DR = CH
NDR = ROWS_PT // DR

SEG = 16
NSEG = CPT // SEG
SEG_U = SEG // 2
NSEG_U = UCPT // SEG_U

NCHUNK_D = EPT_H // CH

BN = 1000

_sc_mesh = plsc.VectorSubcoreMesh(core_axis_name="c", subcore_axis_name="s")


def _mm_body(a_ref, w_ref, o_ref):
    o_ref[...] = lax.dot_general(
        a_ref[...], w_ref[...], (((1,), (1,)), ((), ())),
        preferred_element_type=jnp.float32)


def _mm_bias_body(a_ref, w_ref, b_ref, o_ref):
    o_ref[...] = lax.dot_general(
        a_ref[...], w_ref[...], (((1,), (1,)), ((), ())),
        preferred_element_type=jnp.float32) + b_ref[...]


def _matmul(a, w, bias=None):
    n, k = a.shape
    m = w.shape[0]
    in_specs = [pl.BlockSpec((BN, k), lambda i: (i, 0)),
                pl.BlockSpec((m, k), lambda i: (0, 0))]
    args = [a, w]
    body = _mm_body
    if bias is not None:
        in_specs.append(pl.BlockSpec((1, m), lambda i: (0, 0)))
        args.append(bias.reshape(1, m))
        body = _mm_bias_body
    return pl.pallas_call(
        body,
        grid=(n // BN,),
        in_specs=in_specs,
        out_specs=pl.BlockSpec((BN, m), lambda i: (i, 0)),
        out_shape=jax.ShapeDtypeStruct((n, m), jnp.float32),
    )(*args)


def _conv_body(s2_ref, w_ref, b_ref, xa_ref, s_ref):
    agg = (lax.dot_general(s2_ref[0], w_ref[:, :HALF],
                           (((1,), (1,)), ((), ())),
                           preferred_element_type=jnp.float32)
           + lax.dot_general(s2_ref[1], w_ref[:, HALF:],
                             (((1,), (1,)), ((), ())),
                             preferred_element_type=jnp.float32)
           + b_ref[...])
    xa = jnp.maximum(agg, 0.0)
    xa_ref[...] = xa
    s_ref[...] = jnp.sum(xa * xa, axis=1, keepdims=True)


def _conv(S2, w, b):
    return pl.pallas_call(
        _conv_body,
        grid=(N // BN,),
        in_specs=[pl.BlockSpec((2, BN, HALF), lambda i: (0, i, 0)),
                  pl.BlockSpec((H, H), lambda i: (0, 0)),
                  pl.BlockSpec((1, H), lambda i: (0, 0))],
        out_specs=[pl.BlockSpec((BN, H), lambda i: (i, 0)),
                   pl.BlockSpec((BN, 1), lambda i: (i, 0))],
        out_shape=[jax.ShapeDtypeStruct((N, H), jnp.float32),
                   jax.ShapeDtypeStruct((N, 1), jnp.float32)],
    )(S2, w, b.reshape(1, H))


def _new_h(h_ref, xa_ref, xs_ref, t_ref, s_ref, u0_ref, u1_ref,
           d0_ref, d1_ref):
    xa = xa_ref[...]
    dot = (jnp.sum(xa[:, :HALF] * t_ref[0], axis=1)
           + jnp.sum(xa[:, HALF:] * t_ref[1], axis=1))
    u = u0_ref[:, 0] + u1_ref[:, 0]
    deg = d0_ref[:, 0] + d1_ref[:, 0]
    gsum = deg * s_ref[:, 0] + u - 2.0 * dot
    gamma = jnp.tanh(gsum / (deg + 1e-10))[:, None]
    inv = 1.0 / (1.0 + 2.0 * gamma)
    return (h_ref[...] + gamma * (xa + xs_ref[...])) * inv


def _update_body(h_ref, xa_ref, xs_ref, t_ref, s_ref, u0_ref, u1_ref,
                 d0_ref, d1_ref, o_ref):
    o_ref[...] = _new_h(h_ref, xa_ref, xs_ref, t_ref, s_ref, u0_ref, u1_ref,
                        d0_ref, d1_ref)


def _update_final_body(h_ref, xa_ref, xs_ref, t_ref, s_ref, u0_ref, u1_ref,
                       d0_ref, d1_ref, wfc_ref, bfc_ref, o_ref):
    hb = _new_h(h_ref, xa_ref, xs_ref, t_ref, s_ref, u0_ref, u1_ref,
                d0_ref, d1_ref)
    o_ref[...] = lax.dot_general(
        hb, wfc_ref[...], (((1,), (1,)), ((), ())),
        preferred_element_type=jnp.float32) + bfc_ref[...]


def _update(h, x_agg, x_skip, t2, s, u2, deg2, wfc=None, bfc=None):
    col_spec = pl.BlockSpec((BN, 1), lambda i: (i, 0))
    in_specs = [pl.BlockSpec((BN, H), lambda i: (i, 0)),
                pl.BlockSpec((BN, H), lambda i: (i, 0)),
                pl.BlockSpec((BN, H), lambda i: (i, 0)),
                pl.BlockSpec((2, BN, HALF), lambda i: (0, i, 0)),
                col_spec, col_spec, col_spec, col_spec, col_spec]
    args = [h, x_agg, x_skip, t2, s,
            u2[0, :N, None], u2[1, :N, None],
            deg2[0, :N, None], deg2[1, :N, None]]
    if wfc is None:
        body, d_out = _update_body, H
    else:
        body, d_out = _update_final_body, wfc.shape[0]
        in_specs += [pl.BlockSpec((d_out, H), lambda i: (0, 0)),
                     pl.BlockSpec((1, d_out), lambda i: (0, 0))]
        args += [wfc, bfc.reshape(1, d_out)]
    return pl.pallas_call(
        body,
        grid=(N // BN,),
        in_specs=in_specs,
        out_specs=pl.BlockSpec((BN, d_out), lambda i: (i, 0)),
        out_shape=jax.ShapeDtypeStruct((N, d_out), jnp.float32),
    )(*args)


def _zero_acc(z_hbm, dbuf, acc, r0):
    pltpu.sync_copy(z_hbm, dbuf)
    for j in range(NDR):
        pltpu.sync_copy(dbuf, acc.at[pl.ds(r0 + j * DR, DR)])


def _drain_acc(acc, dbuf, out_hbm, c, r0):
    for j in range(NDR):
        pltpu.sync_copy(acc.at[pl.ds(r0 + j * DR, DR)], dbuf)
        pltpu.sync_copy(dbuf, out_hbm.at[c, pl.ds(r0 + j * DR, DR)])


def _edge_pass(src_hbm, ig_hbm, is_hbm, gidx, sidx, bufs, z_hbm, acc,
               semg, semsc, extra=None):
    def start_g(k, i):
        pltpu.async_copy(src_hbm.at[gidx.at[k]], bufs[i], semg[i])

    def wait_g(i):
        pltpu.make_async_copy(src_hbm.at[gidx.at[0]], bufs[i], semg[i]).wait()

    def start_s(k, i):
        pltpu.async_copy(bufs[i], acc.at[sidx.at[k]], semsc[i], add=True)

    def wait_s(i):
        pltpu.make_async_copy(z_hbm, bufs[i], semsc[i]).wait()

    @pl.loop(0, NSEG)
    def _(g):
        pltpu.sync_copy(ig_hbm.at[g], gidx)
        pltpu.sync_copy(is_hbm.at[g], sidx)
        start_g(0, 0)
        start_g(1, 1)

        @pl.loop(0, SEG, step=4)
        def _(k4):
            for i in range(4):
                jbuf = (i + 2) % 4
                if i < 2:
                    @pl.when((g > 0) | (k4 > 0))
                    def _():
                        wait_s(jbuf)
                else:
                    wait_s(jbuf)

                @pl.when(k4 + i + 2 < SEG)
                def _():
                    start_g(k4 + i + 2, jbuf)

                if extra is not None and i in (0, 2):
                    extra(g, k4, i)
                wait_g(i)
                start_s(k4 + i, i)

    wait_s(2)
    wait_s(3)


def _sc_agg(m2r, rowg_r, cols_r, zmat):
    @functools.partial(
        pl.kernel,
        out_type=jax.ShapeDtypeStruct((2, NP, HALF), jnp.float32),
        mesh=_sc_mesh,
        scratch_types=[
            pltpu.VMEM((SEG, CH), jnp.int32),
            pltpu.VMEM((SEG, CH), jnp.int32),
            pltpu.VMEM((CH, HALF), jnp.float32),
            pltpu.VMEM((CH, HALF), jnp.float32),
            pltpu.VMEM((CH, HALF), jnp.float32),
            pltpu.VMEM((CH, HALF), jnp.float32),
            pltpu.VMEM_SHARED((NP, HALF), jnp.float32),
            pltpu.SemaphoreType.DMA,
            pltpu.SemaphoreType.DMA,
            pltpu.SemaphoreType.DMA,
            pltpu.SemaphoreType.DMA,
            pltpu.SemaphoreType.DMA,
            pltpu.SemaphoreType.DMA,
            pltpu.SemaphoreType.DMA,
            pltpu.SemaphoreType.DMA,
        ],
    )
    def k(m_hbm, rg_hbm, cs_hbm, z_hbm, out_hbm,
          gidx, sidx, b0, b1, b2, b3, acc,
          sg0, sg1, sg2, sg3, ss0, ss1, ss2, ss3):
        c = lax.axis_index("c")
        s = lax.axis_index("s")
        r0 = s * ROWS_PT
        _zero_acc(z_hbm, b0, acc, r0)
        plsc.subcore_barrier()
        _edge_pass(m_hbm, rg_hbm.at[c, s], cs_hbm.at[s],
                   gidx, sidx, (b0, b1, b2, b3), z_hbm, acc,
                   (sg0, sg1, sg2, sg3), (ss0, ss1, ss2, ss3))
        plsc.subcore_barrier()
        _drain_acc(acc, b0, out_hbm, c, r0)

    return k(m2r, rowg_r, cols_r, zmat)


def _sc_gamma(xar, s_vec, colg_r, rows_r, colu_r, rowu_r, zmat, zvec):
    @functools.partial(
        pl.kernel,
        out_type=[jax.ShapeDtypeStruct((2, NP, HALF), jnp.float32),
                  jax.ShapeDtypeStruct((2, NP), jnp.float32)],
        mesh=_sc_mesh,
        scratch_types=[
            pltpu.VMEM((SEG, CH), jnp.int32),
            pltpu.VMEM((SEG, CH), jnp.int32),
            pltpu.VMEM((SEG_U, CH), jnp.int32),
            pltpu.VMEM((SEG_U, CH), jnp.int32),
            pltpu.VMEM((CH, HALF), jnp.float32),
            pltpu.VMEM((CH, HALF), jnp.float32),
            pltpu.VMEM((CH, HALF), jnp.float32),
            pltpu.VMEM((CH, HALF), jnp.float32),
            pltpu.VMEM((CH,), jnp.float32),
            pltpu.VMEM((CH,), jnp.float32),
            pltpu.VMEM((ROWS_PT,), jnp.float32),
            pltpu.VMEM_SHARED((NP, HALF), jnp.float32),
            pltpu.VMEM_SHARED((NP,), jnp.float32),
            pltpu.SemaphoreType.DMA,
            pltpu.SemaphoreType.DMA,
            pltpu.SemaphoreType.DMA,
            pltpu.SemaphoreType.DMA,
            pltpu.SemaphoreType.DMA,
            pltpu.SemaphoreType.DMA,
            pltpu.SemaphoreType.DMA,
            pltpu.SemaphoreType.DMA,
            pltpu.SemaphoreType.DMA,
            pltpu.SemaphoreType.DMA,
        ],
    )
    def k(xa_hbm, s_hbm, cg_hbm, rs_hbm, cu_hbm, ru_hbm, z_hbm, zv_hbm,
          t_hbm, u_hbm,
          gidx, sidx, ucidx, uridx, b0, b1, b2, b3, sva, svb, ubuf,
          acc, uacc,
          sg0, sg1, sg2, sg3, ss0, ss1, ss2, ss3, su0, su1):
        c = lax.axis_index("c")
        s = lax.axis_index("s")
        r0 = s * ROWS_PT
        _zero_acc(z_hbm, b0, acc, r0)
        pltpu.sync_copy(zv_hbm, ubuf)
        pltpu.sync_copy(ubuf, uacc.at[pl.ds(r0, ROWS_PT)])
        plsc.subcore_barrier()

        svals = (sva, svb)
        semu = (su0, su1)

        def u_start(jj, a):
            pltpu.async_copy(s_hbm.at[ucidx.at[jj]], svals[a], semu[a])

        def u_wait(a):
            pltpu.make_async_copy(s_hbm.at[ucidx.at[0]], svals[a],
                                  semu[a]).wait()

        def u_work(g, k4, i):
            j = (k4 + i) // 2
            a = i // 2

            @pl.when(k4 == 0)
            def _():
                if a == 0:
                    pltpu.sync_copy(cu_hbm.at[c, s, g], ucidx)
                    pltpu.sync_copy(ru_hbm.at[c, s, g], uridx)
                    u_start(0, 0)

            @pl.when(j + 1 < SEG_U)
            def _():
                u_start(j + 1, 1 - a)

            u_wait(a)
            pltpu.sync_copy(svals[a], uacc.at[uridx.at[j]], add=True)

        _edge_pass(xa_hbm, cg_hbm.at[c, s], rs_hbm.at[s],
                   gidx, sidx, (b0, b1, b2, b3), z_hbm, acc,
                   (sg0, sg1, sg2, sg3), (ss0, ss1, ss2, ss3),
                   extra=u_work)
        plsc.subcore_barrier()
        _drain_acc(acc, b0, t_hbm, c, r0)
        pltpu.sync_copy(uacc.at[pl.ds(r0, ROWS_PT)], ubuf)
        pltpu.sync_copy(ubuf, u_hbm.at[c, pl.ds(r0, ROWS_PT)])

    return k(xar, s_vec, colg_r, rows_r, colu_r, rowu_r, zmat, zvec)


def _sc_deg(rowh_r, zvec):
    @functools.partial(
        pl.kernel,
        out_type=jax.ShapeDtypeStruct((2, NP), jnp.float32),
        mesh=_sc_mesh,
        scratch_types=[
            pltpu.VMEM((NCHUNK_D, CH), jnp.int32),
            pltpu.VMEM((CH,), jnp.float32),
            pltpu.VMEM((ROWS_PT,), jnp.float32),
            pltpu.VMEM_SHARED((NP,), jnp.float32),
        ],
    )
    def k(rh_hbm, zv_hbm, out_hbm, uridx, ones, zbuf, dacc):
        c = lax.axis_index("c")
        s = lax.axis_index("s")
        r0 = s * ROWS_PT
        for j in range(CH // 16):
            ones[pl.ds(j * 16, 16)] = jnp.ones((16,), jnp.float32)
        pltpu.sync_copy(zv_hbm, zbuf)
        pltpu.sync_copy(zbuf, dacc.at[pl.ds(r0, ROWS_PT)])
        pltpu.sync_copy(rh_hbm.at[c, s], uridx)
        plsc.subcore_barrier()

        @pl.loop(0, NCHUNK_D)
        def _(kk):
            pltpu.sync_copy(ones, dacc.at[uridx.at[kk]], add=True)

        plsc.subcore_barrier()
        pltpu.sync_copy(dacc.at[pl.ds(r0, ROWS_PT)], zbuf)
        pltpu.sync_copy(zbuf, out_hbm.at[c, pl.ds(r0, ROWS_PT)])

    return k(rowh_r, zvec)


def _pad_axis1(a, pad_vals):
    return jnp.concatenate(
        [a, jnp.broadcast_to(pad_vals, a.shape[:-1] + pad_vals.shape[-1:])],
        axis=-1).astype(jnp.int32)


def kernel(x, edge_index, x0, W_in, W_skip, conv_W, conv_b, W_fc, b_fc):
    row = edge_index[0]
    col = edge_index[1]
    two_c = jnp.arange(2, dtype=jnp.int32).reshape(2, 1, 1)

    pad_g = (jnp.arange(PAD_T, dtype=jnp.int32) * 131) % N
    pad_s = N + (jnp.arange(PAD_T, dtype=jnp.int32) % (NP - N))
    pad_gu = pad_g[:PAD_U]
    pad_su = pad_s[:PAD_U]

    row_t = _pad_axis1(row.reshape(NS, EPT), pad_g)
    col_t = _pad_axis1(col.reshape(NS, EPT), pad_g)
    cols_t = _pad_axis1(col.reshape(NS, EPT), pad_s)
    rows_t = _pad_axis1(row.reshape(NS, EPT), pad_s)

    rowg_r = (2 * row_t[None] + two_c).reshape(2, NS, NSEG, SEG, CH)
    colg_r = (2 * col_t[None] + two_c).reshape(2, NS, NSEG, SEG, CH)
    cols_r = cols_t.reshape(NS, NSEG, SEG, CH)
    rows_r = rows_t.reshape(NS, NSEG, SEG, CH)
    colu_r = _pad_axis1(col.reshape(2, NS, EPT_H), pad_gu).reshape(
        2, NS, NSEG_U, SEG_U, CH)
    rowu_r = _pad_axis1(row.reshape(2, NS, EPT_H), pad_su).reshape(
        2, NS, NSEG_U, SEG_U, CH)

    zmat = jnp.zeros((DR, HALF), jnp.float32)
    zvec = jnp.zeros((ROWS_PT,), jnp.float32)

    h = _matmul(x, W_in)
    h0m = _matmul(x0, W_in)
    x_skip = _matmul(h0m, W_skip)
    deg2 = _sc_deg(row.reshape(2, NS, NCHUNK_D, CH), zvec)

    out = None
    for l in range(L):
        S2 = _sc_agg(h.reshape(2 * N, HALF), rowg_r, cols_r, zmat)
        x_agg, s = _conv(S2, conv_W[l], conv_b[l])
        t2, u2 = _sc_gamma(x_agg.reshape(2 * N, HALF), s.reshape(N),
                           colg_r, rows_r, colu_r, rowu_r, zmat, zvec)
        if l < L - 1:
            h = _update(h, x_agg, x_skip, t2, s, u2, deg2)
        else:
            out = _update(h, x_agg, x_skip, t2, s, u2, deg2, W_fc, b_fc)

    return out
